# baseline (speedup 1.0000x reference)
import functools

import jax
import jax.numpy as jnp
from jax.experimental import pallas as pl
from jax.experimental.pallas import tpu as pltpu

NUM_CODES = 8192
DIM = 64
COMMITMENT_COST = 0.25
TN = 1024


CHUNK = 512


def _argmin_body(z_ref, cb2_ref, sumz_ref, sume_ref, idx_ref, mind_ref,
                 dots_ref):
    z = z_ref[0]
    sumz = sumz_ref[0]

    runmin = jnp.full((8, TN), jnp.inf, jnp.float32)
    runv = jnp.zeros((8, TN), jnp.int32)

    for c in range(NUM_CODES // CHUNK):
        buf = c % 2
        dots_ref[buf] = jnp.dot(cb2_ref[c * CHUNK:(c + 1) * CHUNK, :], z,
                                preferred_element_type=jnp.float32)

        def body(i, carry, c=c, buf=buf):
            runmin, runv = carry
            dch = dots_ref[buf, pl.ds(8 * i, 8), :]
            sch = sume_ref[pl.ds(c * CHUNK + 8 * i, 8), :]
            d = (sumz - dch) + sch
            lt = d < runmin
            g = c * (CHUNK // 8) + i
            return jnp.where(lt, d, runmin), jnp.where(lt, g, runv)

        runmin, runv = jax.lax.fori_loop(0, CHUNK // 8, body, (runmin, runv),
                                         unroll=8)

    colmin = jnp.min(runmin, axis=0, keepdims=True)
    srow = jax.lax.broadcasted_iota(jnp.int32, (8, TN), 0)
    cand = jnp.where(runmin == colmin, runv * 8 + srow, NUM_CODES)
    idx_ref[0, 0] = jnp.min(cand, axis=0)
    mind_ref[0, 0] = colmin[0]


@functools.partial(jax.jit, static_argnames=("interpret",))
def _vq(z_e, codebook, interpret=False):
    B, C, H, W = z_e.shape
    N = B * H * W
    HW = H * W
    z2 = z_e.reshape(B, C, HW)
    z_flat = jnp.transpose(z_e, (0, 2, 3, 1)).reshape(-1, C)
    sumz = jnp.sum(z_flat ** 2, axis=1).reshape(B, 1, HW)
    sume = jnp.sum(codebook ** 2, axis=1).reshape(-1, 1)
    cb2 = codebook * 2.0

    idx, mind = pl.pallas_call(
        _argmin_body,
        grid=(B * HW // TN,),
        in_specs=[
            pl.BlockSpec((1, DIM, TN), lambda i: (i, 0, 0)),
            pl.BlockSpec((NUM_CODES, DIM), lambda i: (0, 0)),
            pl.BlockSpec((1, 1, TN), lambda i: (i, 0, 0)),
            pl.BlockSpec((NUM_CODES, 1), lambda i: (0, 0)),
        ],
        out_specs=[
            pl.BlockSpec((1, 1, TN), lambda i: (i, 0, 0)),
            pl.BlockSpec((1, 1, TN), lambda i: (i, 0, 0)),
        ],
        out_shape=[
            jax.ShapeDtypeStruct((N // TN, 1, TN), jnp.int32),
            jax.ShapeDtypeStruct((N // TN, 1, TN), jnp.float32),
        ],
        scratch_shapes=[pltpu.VMEM((2, CHUNK, TN), jnp.float32)],
        interpret=interpret,
    )(z2, cb2, sumz, sume)

    idx = idx.reshape(N)
    z_q_flat = jnp.take(codebook, idx, axis=0)
    z_q_bchw = jnp.transpose(z_q_flat.reshape(B, H, W, C), (0, 3, 1, 2))
    loss = COMMITMENT_COST * (jnp.sum(mind) / (N * C))
    z_q_ste = z_e + jax.lax.stop_gradient(z_q_bchw - z_e)
    return z_q_ste, loss, idx.reshape(B, H, W)


def kernel(z_e, codebook):
    return _vq(z_e, codebook)


# speedup vs baseline: 1.9954x; 1.9954x over previous
import functools

import jax
import jax.numpy as jnp
from jax.experimental import pallas as pl
from jax.experimental.pallas import tpu as pltpu

NUM_CODES = 8192
DIM = 64
COMMITMENT_COST = 0.25
TN = 1024


CHUNK = 512
GROUP = 64


def _argmin_body(z_ref, cb2_ref, sumz_ref, sume_ref, idx_ref, mind_ref):
    z = z_ref[0]
    sumz = sumz_ref[0]
    sume = sume_ref[...]

    runmin = jnp.full((GROUP, TN), jnp.inf, jnp.float32)
    runs = jnp.zeros((GROUP, TN), jnp.int32)

    for c in range(NUM_CODES // CHUNK):
        dotc = jnp.dot(cb2_ref[c * CHUNK:(c + 1) * CHUNK, :], z,
                       preferred_element_type=jnp.float32)
        for i in range(CHUNK // GROUP):
            r0 = i * GROUP
            dch = jax.lax.slice(dotc, (r0, 0), (r0 + GROUP, TN))
            sch = sume[c * CHUNK + r0:c * CHUNK + r0 + GROUP, :]
            d = (sumz - dch) + sch
            lt = d < runmin
            s = c * (CHUNK // GROUP) + i
            runmin = jnp.where(lt, d, runmin)
            runs = jnp.where(lt, s, runs)

    colmin = jnp.min(runmin, axis=0, keepdims=True)
    srow = jax.lax.broadcasted_iota(jnp.int32, (GROUP, TN), 0)
    cand = jnp.where(runmin == colmin, runs * GROUP + srow, NUM_CODES)
    idx_ref[0, 0] = jnp.min(cand, axis=0)
    mind_ref[0, 0] = colmin[0]


@functools.partial(jax.jit, static_argnames=("interpret",))
def _vq(z_e, codebook, interpret=False):
    B, C, H, W = z_e.shape
    N = B * H * W
    HW = H * W
    z2 = z_e.reshape(B, C, HW)
    z_flat = jnp.transpose(z_e, (0, 2, 3, 1)).reshape(-1, C)
    sumz = jnp.sum(z_flat ** 2, axis=1).reshape(B, 1, HW)
    sume = jnp.sum(codebook ** 2, axis=1).reshape(-1, 1)
    cb2 = codebook * 2.0

    idx, mind = pl.pallas_call(
        _argmin_body,
        grid=(B * HW // TN,),
        in_specs=[
            pl.BlockSpec((1, DIM, TN), lambda i: (i, 0, 0)),
            pl.BlockSpec((NUM_CODES, DIM), lambda i: (0, 0)),
            pl.BlockSpec((1, 1, TN), lambda i: (i, 0, 0)),
            pl.BlockSpec((NUM_CODES, 1), lambda i: (0, 0)),
        ],
        out_specs=[
            pl.BlockSpec((1, 1, TN), lambda i: (i, 0, 0)),
            pl.BlockSpec((1, 1, TN), lambda i: (i, 0, 0)),
        ],
        out_shape=[
            jax.ShapeDtypeStruct((N // TN, 1, TN), jnp.int32),
            jax.ShapeDtypeStruct((N // TN, 1, TN), jnp.float32),
        ],
        interpret=interpret,
    )(z2, cb2, sumz, sume)

    idx = idx.reshape(N)
    z_q_flat = jnp.take(codebook, idx, axis=0)
    z_q_bchw = jnp.transpose(z_q_flat.reshape(B, H, W, C), (0, 3, 1, 2))
    loss = COMMITMENT_COST * (jnp.sum(mind) / (N * C))
    z_q_ste = z_e + jax.lax.stop_gradient(z_q_bchw - z_e)
    return z_q_ste, loss, idx.reshape(B, H, W)


def kernel(z_e, codebook):
    return _vq(z_e, codebook)


# speedup vs baseline: 2.3400x; 1.1727x over previous
import functools

import jax
import jax.numpy as jnp
from jax.experimental import pallas as pl
from jax.experimental.pallas import tpu as pltpu
from jax.experimental.pallas import tpu_sc as plsc

NUM_CODES = 8192
DIM = 64
COMMITMENT_COST = 0.25
TN = 1024


CHUNK = 512
GROUP = 64


def _argmin_body(z_ref, cb2_ref, sumz_ref, sume_ref, idx_ref, mind_ref):
    z = z_ref[0]
    sumz = sumz_ref[0]
    sume = sume_ref[...]

    runmin = jnp.full((GROUP, TN), jnp.inf, jnp.float32)
    runs = jnp.zeros((GROUP, TN), jnp.int32)

    for c in range(NUM_CODES // CHUNK):
        dotc = jnp.dot(cb2_ref[c * CHUNK:(c + 1) * CHUNK, :], z,
                       preferred_element_type=jnp.float32)
        for i in range(CHUNK // GROUP):
            r0 = i * GROUP
            dch = jax.lax.slice(dotc, (r0, 0), (r0 + GROUP, TN))
            sch = sume[c * CHUNK + r0:c * CHUNK + r0 + GROUP, :]
            d = (sumz - dch) + sch
            lt = d < runmin
            s = c * (CHUNK // GROUP) + i
            runmin = jnp.where(lt, d, runmin)
            runs = jnp.where(lt, s, runs)

    colmin = jnp.min(runmin, axis=0, keepdims=True)
    srow = jax.lax.broadcasted_iota(jnp.int32, (GROUP, TN), 0)
    cand = jnp.where(runmin == colmin, runs * GROUP + srow, NUM_CODES)
    idx_ref[0, 0] = jnp.min(cand, axis=0)
    mind_ref[0, 0] = colmin[0]


_SC_CORES = 2
_SC_SUBCORES = 16
_SC_WORKERS = _SC_CORES * _SC_SUBCORES


def _sc_gather_body(table_hbm, idx_hbm, out_hbm, idx_v, rows_v, sem):
    bpw = idx_v.shape[0]
    wid = jax.lax.axis_index("s") * _SC_CORES + jax.lax.axis_index("c")
    base = wid * bpw
    pltpu.sync_copy(idx_hbm.at[pl.ds(base, bpw)], idx_v)
    pltpu.async_copy(table_hbm.at[idx_v], rows_v, sem).wait()
    pltpu.sync_copy(rows_v, out_hbm.at[pl.ds(base, bpw)])


def _sc_gather(codebook, idx_flat):
    n = idx_flat.shape[0]
    bpw = n // _SC_WORKERS
    return pl.kernel(
        _sc_gather_body,
        out_type=jax.ShapeDtypeStruct((n, codebook.shape[1]), jnp.float32),
        mesh=plsc.VectorSubcoreMesh(
            core_axis_name="c", subcore_axis_name="s",
            num_cores=_SC_CORES, num_subcores=_SC_SUBCORES),
        scratch_types=[
            pltpu.VMEM((bpw,), jnp.int32),
            pltpu.VMEM((bpw, codebook.shape[1]), jnp.float32),
            pltpu.SemaphoreType.DMA,
        ],
    )(codebook, idx_flat)


@functools.partial(jax.jit, static_argnames=("interpret",))
def _vq(z_e, codebook, interpret=False):
    B, C, H, W = z_e.shape
    N = B * H * W
    HW = H * W
    z2 = z_e.reshape(B, C, HW)
    z_flat = jnp.transpose(z_e, (0, 2, 3, 1)).reshape(-1, C)
    sumz = jnp.sum(z_flat ** 2, axis=1).reshape(B, 1, HW)
    sume = jnp.sum(codebook ** 2, axis=1).reshape(-1, 1)
    cb2 = codebook * 2.0

    idx, mind = pl.pallas_call(
        _argmin_body,
        grid=(B * HW // TN,),
        in_specs=[
            pl.BlockSpec((1, DIM, TN), lambda i: (i, 0, 0)),
            pl.BlockSpec((NUM_CODES, DIM), lambda i: (0, 0)),
            pl.BlockSpec((1, 1, TN), lambda i: (i, 0, 0)),
            pl.BlockSpec((NUM_CODES, 1), lambda i: (0, 0)),
        ],
        out_specs=[
            pl.BlockSpec((1, 1, TN), lambda i: (i, 0, 0)),
            pl.BlockSpec((1, 1, TN), lambda i: (i, 0, 0)),
        ],
        out_shape=[
            jax.ShapeDtypeStruct((N // TN, 1, TN), jnp.int32),
            jax.ShapeDtypeStruct((N // TN, 1, TN), jnp.float32),
        ],
        interpret=interpret,
    )(z2, cb2, sumz, sume)

    idx = idx.reshape(N)
    if interpret:
        z_q_flat = jnp.take(codebook, idx, axis=0)
    else:
        cb_pad = jnp.pad(codebook, ((0, 0), (0, 128 - DIM)))
        z_q_flat = _sc_gather(cb_pad, idx)[:, :DIM]
    z_q_bchw = jnp.transpose(z_q_flat.reshape(B, H, W, C), (0, 3, 1, 2))
    loss = COMMITMENT_COST * (jnp.sum(mind) / (N * C))
    z_q_ste = z_e + jax.lax.stop_gradient(z_q_bchw - z_e)
    return z_q_ste, loss, idx.reshape(B, H, W)


def kernel(z_e, codebook):
    return _vq(z_e, codebook)


# speedup vs baseline: 2.7560x; 1.1778x over previous
import functools

import jax
import jax.numpy as jnp
from jax.experimental import pallas as pl
from jax.experimental.pallas import tpu as pltpu
from jax.experimental.pallas import tpu_sc as plsc

NUM_CODES = 8192
DIM = 64
COMMITMENT_COST = 0.25
TN = 1024


CHUNK = 512
GROUP = 8


def _argmin_body(z_ref, cb2_ref, sumz_ref, sume_ref, idx_ref, mind_ref):
    z = z_ref[0]
    sumz = sumz_ref[0]
    sume = sume_ref[...]

    runmin = jnp.full((GROUP, TN), jnp.inf, jnp.float32)
    runs = jnp.zeros((GROUP, TN), jnp.int32)

    for c in range(NUM_CODES // CHUNK):
        dotc = jnp.dot(cb2_ref[c * CHUNK:(c + 1) * CHUNK, :], z,
                       preferred_element_type=jnp.float32)
        for i in range(CHUNK // GROUP):
            r0 = i * GROUP
            dch = jax.lax.slice(dotc, (r0, 0), (r0 + GROUP, TN))
            sch = sume[c * CHUNK + r0:c * CHUNK + r0 + GROUP, :]
            d = (sumz - dch) + sch
            lt = d < runmin
            s = c * (CHUNK // GROUP) + i
            runmin = jnp.where(lt, d, runmin)
            runs = jnp.where(lt, s, runs)

    colmin = jnp.min(runmin, axis=0, keepdims=True)
    srow = jax.lax.broadcasted_iota(jnp.int32, (GROUP, TN), 0)
    cand = jnp.where(runmin == colmin, runs * GROUP + srow, NUM_CODES)
    idx_ref[0, 0] = jnp.min(cand, axis=0)
    mind_ref[0, 0] = colmin[0]


_SC_CORES = 2
_SC_SUBCORES = 16
_SC_WORKERS = _SC_CORES * _SC_SUBCORES


def _sc_gather_body(table_hbm, idx_hbm, out_hbm, idx_v, rows_v, sem):
    bpw = idx_v.shape[0]
    wid = jax.lax.axis_index("s") * _SC_CORES + jax.lax.axis_index("c")
    base = wid * bpw
    pltpu.sync_copy(idx_hbm.at[pl.ds(base, bpw)], idx_v)
    pltpu.async_copy(table_hbm.at[idx_v], rows_v, sem).wait()
    pltpu.sync_copy(rows_v, out_hbm.at[pl.ds(base, bpw)])


def _sc_gather(codebook, idx_flat):
    n = idx_flat.shape[0]
    bpw = n // _SC_WORKERS
    return pl.kernel(
        _sc_gather_body,
        out_type=jax.ShapeDtypeStruct((n, codebook.shape[1]), jnp.float32),
        mesh=plsc.VectorSubcoreMesh(
            core_axis_name="c", subcore_axis_name="s",
            num_cores=_SC_CORES, num_subcores=_SC_SUBCORES),
        scratch_types=[
            pltpu.VMEM((bpw,), jnp.int32),
            pltpu.VMEM((bpw, codebook.shape[1]), jnp.float32),
            pltpu.SemaphoreType.DMA,
        ],
    )(codebook, idx_flat)


@functools.partial(jax.jit, static_argnames=("interpret",))
def _vq(z_e, codebook, interpret=False):
    B, C, H, W = z_e.shape
    N = B * H * W
    HW = H * W
    z2 = z_e.reshape(B, C, HW)
    z_flat = jnp.transpose(z_e, (0, 2, 3, 1)).reshape(-1, C)
    sumz = jnp.sum(z_flat ** 2, axis=1).reshape(B, 1, HW)
    sume = jnp.sum(codebook ** 2, axis=1).reshape(-1, 1)
    cb2 = codebook * 2.0

    idx, mind = pl.pallas_call(
        _argmin_body,
        grid=(B * HW // TN,),
        in_specs=[
            pl.BlockSpec((1, DIM, TN), lambda i: (i, 0, 0)),
            pl.BlockSpec((NUM_CODES, DIM), lambda i: (0, 0)),
            pl.BlockSpec((1, 1, TN), lambda i: (i, 0, 0)),
            pl.BlockSpec((NUM_CODES, 1), lambda i: (0, 0)),
        ],
        out_specs=[
            pl.BlockSpec((1, 1, TN), lambda i: (i, 0, 0)),
            pl.BlockSpec((1, 1, TN), lambda i: (i, 0, 0)),
        ],
        out_shape=[
            jax.ShapeDtypeStruct((N // TN, 1, TN), jnp.int32),
            jax.ShapeDtypeStruct((N // TN, 1, TN), jnp.float32),
        ],
        interpret=interpret,
    )(z2, cb2, sumz, sume)

    idx = idx.reshape(N)
    if interpret:
        z_q_flat = jnp.take(codebook, idx, axis=0)
    else:
        cb_pad = jnp.pad(codebook, ((0, 0), (0, 128 - DIM)))
        z_q_flat = _sc_gather(cb_pad, idx)[:, :DIM]
    z_q_bchw = jnp.transpose(z_q_flat.reshape(B, H, W, C), (0, 3, 1, 2))
    loss = COMMITMENT_COST * (jnp.sum(mind) / (N * C))
    z_q_ste = z_e + jax.lax.stop_gradient(z_q_bchw - z_e)
    return z_q_ste, loss, idx.reshape(B, H, W)


def kernel(z_e, codebook):
    return _vq(z_e, codebook)
